# baseline (speedup 1.0000x reference)
import functools

import jax
import jax.numpy as jnp
from jax import lax
from jax.experimental import pallas as pl
from jax.experimental.pallas import tpu as pltpu
from jax.experimental.pallas import tpu_sc as plsc

N = 10000
HEADS = 8
D = 128
G = 64
NW = 32
NSUB = 16
B = 128
NACC = 10112
RPT = NACC // NSUB
R = 400


def _mm_alpha_body(x_ref, w_ref, asf_ref, adf_ref, sel_ref, h_ref, ta_ref, tb_ref):
    h = jnp.dot(x_ref[...], w_ref[...], preferred_element_type=jnp.float32)
    h_ref[...] = h
    ta_ref[...] = jnp.dot(h * asf_ref[...], sel_ref[...],
                          preferred_element_type=jnp.float32)
    tb_ref[...] = jnp.dot(h * adf_ref[...], sel_ref[...],
                          preferred_element_type=jnp.float32)


def _combine2_body(acc_ref, den_ref, b1_ref, w2_ref, a2s_ref, a2d_ref,
                   h2_ref, ta_ref, tb_ref):
    den = den_ref[0] + den_ref[1]
    h2 = jnp.zeros((R, D), jnp.float32)
    for h in range(HEADS):
        a = acc_ref[0, h] + acc_ref[1, h]
        x2 = jnp.maximum(a / (den[:, h:h + 1] + 1e-16) + b1_ref[h][None, :], 0.0)
        h2 = h2 + jnp.dot(x2, w2_ref[h], preferred_element_type=jnp.float32)
    h2_ref[...] = h2
    ta_ref[...] = jnp.dot(h2, a2s_ref[...], preferred_element_type=jnp.float32)
    tb_ref[...] = jnp.dot(h2, a2d_ref[...], preferred_element_type=jnp.float32)


def _combine1_body(acc_ref, den_ref, b2_ref, out_ref):
    den = den_ref[0] + den_ref[1]
    a = acc_ref[0, 0] + acc_ref[1, 0]
    out_ref[...] = jnp.maximum(a / (den[:, 0:1] + 1e-16) + b2_ref[...], 0.0)


def _pool_fc_body(x_ref, bt_ref, wfc_ref, bfc_ref, out_ref):
    pid = pl.program_id(0)
    x = x_ref[...]
    bt = bt_ref[...]
    rows = []
    for gl in range(8):
        g = pid * 8 + gl
        m = jnp.where(bt == g, x, -jnp.inf)
        rows.append(jnp.max(m, axis=0, keepdims=True))
    pooled = jnp.concatenate(rows, axis=0)
    pooled = jnp.where(pooled == -jnp.inf, 0.0, pooled)
    out_ref[...] = jnp.maximum(
        jnp.dot(pooled, wfc_ref[...], preferred_element_type=jnp.float32)
        + bfc_ref[...], 0.0)


def _mm_alpha(x, w, asf, adf, sel):
    n = x.shape[0]
    grid = n // R
    return pl.pallas_call(
        _mm_alpha_body,
        grid=(grid,),
        in_specs=[
            pl.BlockSpec((R, x.shape[1]), lambda i: (i, 0)),
            pl.BlockSpec(w.shape, lambda i: (0, 0)),
            pl.BlockSpec(asf.shape, lambda i: (0, 0)),
            pl.BlockSpec(adf.shape, lambda i: (0, 0)),
            pl.BlockSpec(sel.shape, lambda i: (0, 0)),
        ],
        out_specs=[
            pl.BlockSpec((R, w.shape[1]), lambda i: (i, 0)),
            pl.BlockSpec((R, 16), lambda i: (i, 0)),
            pl.BlockSpec((R, 16), lambda i: (i, 0)),
        ],
        out_shape=[
            jax.ShapeDtypeStruct((n, w.shape[1]), jnp.float32),
            jax.ShapeDtypeStruct((n, 16), jnp.float32),
            jax.ShapeDtypeStruct((n, 16), jnp.float32),
        ],
    )(x, w, asf, adf, sel)


def _combine2(acc, den, b1r, w2r, a2s, a2d):
    grid = N // R
    return pl.pallas_call(
        _combine2_body,
        grid=(grid,),
        in_specs=[
            pl.BlockSpec((2, HEADS, R, D), lambda i: (0, 0, i, 0)),
            pl.BlockSpec((2, R, 16), lambda i: (0, i, 0)),
            pl.BlockSpec((HEADS, D), lambda i: (0, 0)),
            pl.BlockSpec((HEADS, D, D), lambda i: (0, 0, 0)),
            pl.BlockSpec((D, 16), lambda i: (0, 0)),
            pl.BlockSpec((D, 16), lambda i: (0, 0)),
        ],
        out_specs=[
            pl.BlockSpec((R, D), lambda i: (i, 0)),
            pl.BlockSpec((R, 16), lambda i: (i, 0)),
            pl.BlockSpec((R, 16), lambda i: (i, 0)),
        ],
        out_shape=[
            jax.ShapeDtypeStruct((N, D), jnp.float32),
            jax.ShapeDtypeStruct((N, 16), jnp.float32),
            jax.ShapeDtypeStruct((N, 16), jnp.float32),
        ],
    )(acc, den, b1r, w2r, a2s, a2d)


def _combine1(acc, den, b2r):
    grid = N // R
    return pl.pallas_call(
        _combine1_body,
        grid=(grid,),
        in_specs=[
            pl.BlockSpec((2, 1, R, D), lambda i: (0, 0, i, 0)),
            pl.BlockSpec((2, R, 16), lambda i: (0, i, 0)),
            pl.BlockSpec((1, D), lambda i: (0, 0)),
        ],
        out_specs=pl.BlockSpec((R, D), lambda i: (i, 0)),
        out_shape=jax.ShapeDtypeStruct((N, D), jnp.float32),
    )(acc, den, b2r)


def _pool_fc(x, bt, wfc, bfcr):
    return pl.pallas_call(
        _pool_fc_body,
        grid=(G // 8,),
        in_specs=[
            pl.BlockSpec((N, D), lambda i: (0, 0)),
            pl.BlockSpec((N, 1), lambda i: (0, 0)),
            pl.BlockSpec((D, D), lambda i: (0, 0)),
            pl.BlockSpec((1, D), lambda i: (0, 0)),
        ],
        out_specs=pl.BlockSpec((8, D), lambda i: (i, 0)),
        out_shape=jax.ShapeDtypeStruct((G, D), jnp.float32),
    )(x, bt, wfc, bfcr)


def _make_exdenom(e2p):
    pt = e2p // NW
    nb = pt // B
    mesh = plsc.VectorSubcoreMesh(core_axis_name="c", subcore_axis_name="s")

    @functools.partial(
        pl.kernel, mesh=mesh,
        compiler_params=pltpu.CompilerParams(use_tc_tiling_on_sc=False),
        out_type=[
            jax.ShapeDtypeStruct((e2p, 16), jnp.float32),
            jax.ShapeDtypeStruct((2, NACC, 16), jnp.float32),
        ],
        scratch_types=[
            pltpu.VMEM((B,), jnp.int32),
            pltpu.VMEM((B,), jnp.int32),
            pltpu.VMEM((B,), jnp.int32),
            pltpu.VMEM((B,), jnp.int32),
            pltpu.VMEM((B, 16), jnp.float32),
            pltpu.VMEM((B, 16), jnp.float32),
            pltpu.VMEM((B, 16), jnp.float32),
            pltpu.VMEM((B, 16), jnp.float32),
            pltpu.VMEM((B, 16), jnp.float32),
            pltpu.VMEM((B, 16), jnp.float32),
            pltpu.VMEM_SHARED((NACC, 16), jnp.float32),
            pltpu.SemaphoreType.DMA,
            pltpu.SemaphoreType.DMA,
            pltpu.SemaphoreType.DMA,
            pltpu.SemaphoreType.DMA,
            pltpu.SemaphoreType.DMA,
            pltpu.SemaphoreType.DMA,
        ],
    )
    def exdenom(src_h, dst_h, ta_h, tb_h, zer_h, ex_h, denp_h,
                sidx_a, sidx_b, didx_a, didx_b, abuf_a, abuf_b,
                bbuf_a, bbuf_b, exbuf_a, exbuf_b, den_sh,
                asem_a, asem_b, bsem_a, bsem_b, dsem_a, dsem_b):
        cid = lax.axis_index("c")
        tid = lax.axis_index("s")
        wid = tid * 2 + cid
        pltpu.sync_copy(zer_h.at[pl.ds(tid * RPT, RPT)],
                        den_sh.at[pl.ds(tid * RPT, RPT)])
        plsc.subcore_barrier()
        base = wid * pt

        bufs = {
            0: (sidx_a, didx_a, abuf_a, bbuf_a, exbuf_a, asem_a, bsem_a,
                dsem_a),
            1: (sidx_b, didx_b, abuf_b, bbuf_b, exbuf_b, asem_b, bsem_b,
                dsem_b),
        }

        def start(bi, t):
            sidx, didx, abuf, bbuf, exbuf, asem, bsem, dsem = bufs[bi]
            off = base + t * B
            pltpu.sync_copy(src_h.at[pl.ds(off, B)], sidx)
            pltpu.sync_copy(dst_h.at[pl.ds(off, B)], didx)
            pltpu.async_copy(ta_h.at[sidx], abuf, asem)
            pltpu.async_copy(tb_h.at[didx], bbuf, bsem)

        def finish(bi, t):
            sidx, didx, abuf, bbuf, exbuf, asem, bsem, dsem = bufs[bi]
            off = base + t * B
            pltpu.make_async_copy(ta_h.at[sidx], abuf, asem).wait()
            pltpu.make_async_copy(tb_h.at[didx], bbuf, bsem).wait()

            @plsc.parallel_loop(0, B, unroll=8)
            def cbody(k):
                v = abuf[k] + bbuf[k]
                v = jnp.maximum(v, 0.2 * v)
                exbuf[k] = jnp.exp(v)
            pltpu.sync_copy(exbuf, ex_h.at[pl.ds(off, B)])
            pltpu.sync_copy(exbuf, den_sh.at[didx], add=True)

        start(0, 0)

        def pair_body(p, carry):
            t0 = 2 * p
            start(1, t0 + 1)
            finish(0, t0)
            start(0, t0 + 2)
            finish(1, t0 + 1)
            return carry

        lax.fori_loop(0, (nb - 1) // 2, pair_body, 0)
        finish(0, nb - 1)
        plsc.subcore_barrier()
        pltpu.sync_copy(den_sh.at[pl.ds(tid * RPT, RPT)],
                        denp_h.at[cid, pl.ds(tid * RPT, RPT)])

    return exdenom


def _make_accum(e2p, nheads, nb0):
    nbt = e2p // (NSUB * B)
    nb1 = nbt - nb0
    mesh = plsc.VectorSubcoreMesh(core_axis_name="c", subcore_axis_name="s")

    @functools.partial(
        pl.kernel, mesh=mesh,
        compiler_params=pltpu.CompilerParams(use_tc_tiling_on_sc=False),
        out_type=jax.ShapeDtypeStruct((2, nheads, NACC, D), jnp.float32),
        scratch_types=[
            pltpu.VMEM((max(nb0, nb1) * B,), jnp.int32),
            pltpu.VMEM((B,), jnp.int32),
            pltpu.VMEM((B,), jnp.int32),
            pltpu.VMEM((B,), jnp.int32),
            pltpu.VMEM((B,), jnp.int32),
            pltpu.VMEM((B, 16), jnp.float32),
            pltpu.VMEM((B, 16), jnp.float32),
            pltpu.VMEM((B, D), jnp.float32),
            pltpu.VMEM((B, D), jnp.float32),
            pltpu.VMEM_SHARED((NACC, D), jnp.float32),
            pltpu.SemaphoreType.DMA,
            pltpu.SemaphoreType.DMA,
            pltpu.SemaphoreType.DMA,
            pltpu.SemaphoreType.DMA,
            pltpu.SemaphoreType.DMA,
            pltpu.SemaphoreType.DMA,
            pltpu.SemaphoreType.DMA,
            pltpu.SemaphoreType.DMA,
        ],
    )
    def accum(src_h, dst_h, ex_h, hflat_h, zer_h, acc_h,
              sidx_all, didx_a, didx_b, gidx_a, gidx_b, exb_a, exb_b,
              rows_a, rows_b, acc_sh,
              gsem_a, gsem_b, xsem_a, xsem_b, dsem_a, dsem_b,
              ssem_a, ssem_b):
        cid = lax.axis_index("c")
        tid = lax.axis_index("s")
        nbc = jnp.where(cid == 0, nb0, nb1)
        base = jnp.where(cid == 0, tid * (nb0 * B),
                         NSUB * nb0 * B + tid * (nb1 * B))
        pltpu.sync_copy(src_h.at[pl.ds(base, max(nb0, nb1) * B)], sidx_all)

        bufs = {
            0: (gidx_a, didx_a, exb_a, rows_a, gsem_a, xsem_a, dsem_a,
                ssem_a),
            1: (gidx_b, didx_b, exb_b, rows_b, gsem_b, xsem_b, dsem_b,
                ssem_b),
        }

        def start(bi, t, hh):
            gidx, didx, exb, rows, gsem, xsem, dsem, ssem = bufs[bi]
            for g in range(B // 16):
                s16 = sidx_all[pl.ds(t * B + g * 16, 16)]
                if nheads > 1:
                    gidx[pl.ds(g * 16, 16)] = s16 * nheads + hh
                else:
                    gidx[pl.ds(g * 16, 16)] = s16
            pltpu.async_copy(hflat_h.at[gidx], rows, gsem)
            pltpu.async_copy(ex_h.at[pl.ds(base + t * B, B)], exb, xsem)
            pltpu.async_copy(dst_h.at[pl.ds(base + t * B, B)], didx, dsem)

        def mul(bi, t, hh):
            gidx, didx, exb, rows, gsem, xsem, dsem, ssem = bufs[bi]
            pltpu.make_async_copy(hflat_h.at[gidx], rows, gsem).wait()
            pltpu.make_async_copy(ex_h.at[pl.ds(base + t * B, B)], exb,
                                  xsem).wait()
            pltpu.make_async_copy(dst_h.at[pl.ds(base + t * B, B)], didx,
                                  dsem).wait()

            @plsc.parallel_loop(0, B, unroll=8)
            def wbody(j):
                wl = jnp.full((16,), exb[j][hh], jnp.float32)
                for c in range(D // 16):
                    rows[j, pl.ds(c * 16, 16)] = (
                        rows[j, pl.ds(c * 16, 16)] * wl)

        def finish(bi, t, hh):
            gidx, didx, exb, rows, gsem, xsem, dsem, ssem = bufs[bi]
            mul(bi, t, hh)
            pltpu.sync_copy(rows, acc_sh.at[didx], add=True)

        for hh in range(nheads):
            pltpu.sync_copy(zer_h.at[pl.ds(tid * RPT, RPT)],
                            acc_sh.at[pl.ds(tid * RPT, RPT)])
            plsc.subcore_barrier()
            start(0, 0, hh)

            def pair_body(p, c1):
                t0 = 2 * p
                start(1, t0 + 1, hh)
                finish(0, t0, hh)
                start(0, t0 + 2, hh)
                finish(1, t0 + 1, hh)
                return c1

            lax.fori_loop(0, (nbc - 1) // 2, pair_body, 0)
            finish(0, nbc - 1, hh)
            plsc.subcore_barrier()
            pltpu.sync_copy(acc_sh.at[pl.ds(tid * RPT, RPT)],
                            acc_h.at[cid, hh, pl.ds(tid * RPT, RPT)])
            plsc.subcore_barrier()

    return accum


def kernel(x, edge_index, batch, W1, a_src1, a_dst1, b1, W2, a_src2, a_dst2,
           b2, Wfc, bfc):
    n = x.shape[0]
    e = edge_index.shape[1]
    e2 = e + n
    e2p = -(-e2 // (B * NW)) * (B * NW)
    nbt = e2p // (NSUB * B)
    nb0 = 65
    pad_extra = max(0, (2 * nb0 - nbt)) * B

    loop = jnp.arange(n, dtype=edge_index.dtype)
    src = jnp.concatenate([edge_index[0], loop,
                           jnp.zeros((e2p - e2 + pad_extra,),
                                     edge_index.dtype)])
    dst = jnp.concatenate([edge_index[1], loop,
                           jnp.full((e2p - e2 + pad_extra,), n,
                                    edge_index.dtype)])

    eye = jnp.eye(HEADS, dtype=jnp.float32)
    sel1 = jnp.concatenate(
        [jnp.repeat(eye, D, axis=0), jnp.zeros((HEADS * D, 8), jnp.float32)],
        axis=1)
    a2s = jnp.concatenate([a_src2.T, jnp.zeros((D, 15), jnp.float32)], axis=1)
    a2d = jnp.concatenate([a_dst2.T, jnp.zeros((D, 15), jnp.float32)], axis=1)

    zer_den = jnp.zeros((NACC, 16), jnp.float32)
    zer_acc = jnp.zeros((NACC, D), jnp.float32)
    padrows = jnp.zeros((NACC - n, 16), jnp.float32)

    h1, ta1, tb1 = _mm_alpha(x, W1, a_src1.reshape(1, -1),
                             a_dst1.reshape(1, -1), sel1)
    ta1p = jnp.concatenate([ta1, padrows])
    tb1p = jnp.concatenate([tb1, padrows])
    ex1, denp1 = _make_exdenom(e2p)(src, dst, ta1p, tb1p, zer_den)
    acc1 = _make_accum(e2p, HEADS, nb0)(src, dst, ex1,
                                        h1.reshape(n * HEADS, D), zer_acc)

    h2, ta2, tb2 = _combine2(acc1, denp1, b1.reshape(HEADS, D),
                             W2.reshape(HEADS, D, D), a2s, a2d)
    ta2p = jnp.concatenate([ta2, padrows])
    tb2p = jnp.concatenate([tb2, padrows])
    ex2, denp2 = _make_exdenom(e2p)(src, dst, ta2p, tb2p, zer_den)
    acc2 = _make_accum(e2p, 1, nb0)(src, dst, ex2, h2, zer_acc)

    out2 = _combine1(acc2, denp2, b2.reshape(1, D))

    return _pool_fc(out2, batch.reshape(n, 1).astype(jnp.int32), Wfc,
                    bfc.reshape(1, D))


# speedup vs baseline: 1.1719x; 1.1719x over previous
import functools

import jax
import jax.numpy as jnp
from jax import lax
from jax.experimental import pallas as pl
from jax.experimental.pallas import tpu as pltpu
from jax.experimental.pallas import tpu_sc as plsc

N = 10000
HEADS = 8
D = 128
G = 64
NW = 32
NSUB = 16
B = 128
NACC = 10112
RPT = NACC // NSUB
R = 400


def _mm_alpha_body(x_ref, w_ref, asf_ref, adf_ref, sel_ref, h_ref, ta_ref, tb_ref):
    h = jnp.dot(x_ref[...], w_ref[...], preferred_element_type=jnp.float32)
    h_ref[...] = h
    ta_ref[...] = jnp.dot(h * asf_ref[...], sel_ref[...],
                          preferred_element_type=jnp.float32)
    tb_ref[...] = jnp.dot(h * adf_ref[...], sel_ref[...],
                          preferred_element_type=jnp.float32)


def _combine2_body(acc_ref, den_ref, b1_ref, w2_ref, a2s_ref, a2d_ref,
                   h2_ref, ta_ref, tb_ref):
    den = den_ref[0] + den_ref[1]
    h2 = jnp.zeros((R, D), jnp.float32)
    for h in range(HEADS):
        a = acc_ref[0, h] + acc_ref[1, h]
        x2 = jnp.maximum(a / (den[:, h:h + 1] + 1e-16) + b1_ref[h][None, :], 0.0)
        h2 = h2 + jnp.dot(x2, w2_ref[h], preferred_element_type=jnp.float32)
    h2_ref[...] = h2
    ta_ref[...] = jnp.dot(h2, a2s_ref[...], preferred_element_type=jnp.float32)
    tb_ref[...] = jnp.dot(h2, a2d_ref[...], preferred_element_type=jnp.float32)


def _combine1_body(acc_ref, den_ref, b2_ref, out_ref):
    den = den_ref[0] + den_ref[1]
    a = acc_ref[0, 0] + acc_ref[1, 0]
    out_ref[...] = jnp.maximum(a / (den[:, 0:1] + 1e-16) + b2_ref[...], 0.0)


def _pool_fc_body(x_ref, bt_ref, wfc_ref, bfc_ref, out_ref):
    pid = pl.program_id(0)
    x = x_ref[...]
    bt = bt_ref[...]
    rows = []
    for gl in range(8):
        g = pid * 8 + gl
        m = jnp.where(bt == g, x, -jnp.inf)
        rows.append(jnp.max(m, axis=0, keepdims=True))
    pooled = jnp.concatenate(rows, axis=0)
    pooled = jnp.where(pooled == -jnp.inf, 0.0, pooled)
    out_ref[...] = jnp.maximum(
        jnp.dot(pooled, wfc_ref[...], preferred_element_type=jnp.float32)
        + bfc_ref[...], 0.0)


def _mm_alpha(x, w, asf, adf, sel):
    n = x.shape[0]
    grid = n // R
    return pl.pallas_call(
        _mm_alpha_body,
        grid=(grid,),
        in_specs=[
            pl.BlockSpec((R, x.shape[1]), lambda i: (i, 0)),
            pl.BlockSpec(w.shape, lambda i: (0, 0)),
            pl.BlockSpec(asf.shape, lambda i: (0, 0)),
            pl.BlockSpec(adf.shape, lambda i: (0, 0)),
            pl.BlockSpec(sel.shape, lambda i: (0, 0)),
        ],
        out_specs=[
            pl.BlockSpec((R, w.shape[1]), lambda i: (i, 0)),
            pl.BlockSpec((R, 16), lambda i: (i, 0)),
            pl.BlockSpec((R, 16), lambda i: (i, 0)),
        ],
        out_shape=[
            jax.ShapeDtypeStruct((n, w.shape[1]), jnp.float32),
            jax.ShapeDtypeStruct((n, 16), jnp.float32),
            jax.ShapeDtypeStruct((n, 16), jnp.float32),
        ],
    )(x, w, asf, adf, sel)


def _combine2(acc, den, b1r, w2r, a2s, a2d):
    grid = N // R
    return pl.pallas_call(
        _combine2_body,
        grid=(grid,),
        in_specs=[
            pl.BlockSpec((2, HEADS, R, D), lambda i: (0, 0, i, 0)),
            pl.BlockSpec((2, R, 16), lambda i: (0, i, 0)),
            pl.BlockSpec((HEADS, D), lambda i: (0, 0)),
            pl.BlockSpec((HEADS, D, D), lambda i: (0, 0, 0)),
            pl.BlockSpec((D, 16), lambda i: (0, 0)),
            pl.BlockSpec((D, 16), lambda i: (0, 0)),
        ],
        out_specs=[
            pl.BlockSpec((R, D), lambda i: (i, 0)),
            pl.BlockSpec((R, 16), lambda i: (i, 0)),
            pl.BlockSpec((R, 16), lambda i: (i, 0)),
        ],
        out_shape=[
            jax.ShapeDtypeStruct((N, D), jnp.float32),
            jax.ShapeDtypeStruct((N, 16), jnp.float32),
            jax.ShapeDtypeStruct((N, 16), jnp.float32),
        ],
    )(acc, den, b1r, w2r, a2s, a2d)


def _combine1(acc, den, b2r):
    grid = N // R
    return pl.pallas_call(
        _combine1_body,
        grid=(grid,),
        in_specs=[
            pl.BlockSpec((2, 1, R, D), lambda i: (0, 0, i, 0)),
            pl.BlockSpec((2, R, 16), lambda i: (0, i, 0)),
            pl.BlockSpec((1, D), lambda i: (0, 0)),
        ],
        out_specs=pl.BlockSpec((R, D), lambda i: (i, 0)),
        out_shape=jax.ShapeDtypeStruct((N, D), jnp.float32),
    )(acc, den, b2r)


def _pool_fc(x, bt, wfc, bfcr):
    return pl.pallas_call(
        _pool_fc_body,
        grid=(G // 8,),
        in_specs=[
            pl.BlockSpec((N, D), lambda i: (0, 0)),
            pl.BlockSpec((N, 1), lambda i: (0, 0)),
            pl.BlockSpec((D, D), lambda i: (0, 0)),
            pl.BlockSpec((1, D), lambda i: (0, 0)),
        ],
        out_specs=pl.BlockSpec((8, D), lambda i: (i, 0)),
        out_shape=jax.ShapeDtypeStruct((G, D), jnp.float32),
    )(x, bt, wfc, bfcr)


def _make_exdenom(e2p):
    pt = e2p // NW
    nb = pt // B
    mesh = plsc.VectorSubcoreMesh(core_axis_name="c", subcore_axis_name="s")

    @functools.partial(
        pl.kernel, mesh=mesh,
        compiler_params=pltpu.CompilerParams(use_tc_tiling_on_sc=False),
        out_type=[
            jax.ShapeDtypeStruct((e2p, 16), jnp.float32),
            jax.ShapeDtypeStruct((2, NACC, 16), jnp.float32),
        ],
        scratch_types=[
            pltpu.VMEM((B,), jnp.int32),
            pltpu.VMEM((B,), jnp.int32),
            pltpu.VMEM((B,), jnp.int32),
            pltpu.VMEM((B,), jnp.int32),
            pltpu.VMEM((B, 16), jnp.float32),
            pltpu.VMEM((B, 16), jnp.float32),
            pltpu.VMEM((B, 16), jnp.float32),
            pltpu.VMEM((B, 16), jnp.float32),
            pltpu.VMEM((B, 16), jnp.float32),
            pltpu.VMEM((B, 16), jnp.float32),
            pltpu.VMEM_SHARED((NACC, 16), jnp.float32),
            pltpu.SemaphoreType.DMA,
            pltpu.SemaphoreType.DMA,
            pltpu.SemaphoreType.DMA,
            pltpu.SemaphoreType.DMA,
            pltpu.SemaphoreType.DMA,
            pltpu.SemaphoreType.DMA,
        ],
    )
    def exdenom(src_h, dst_h, ta_h, tb_h, zer_h, ex_h, denp_h,
                sidx_a, sidx_b, didx_a, didx_b, abuf_a, abuf_b,
                bbuf_a, bbuf_b, exbuf_a, exbuf_b, den_sh,
                asem_a, asem_b, bsem_a, bsem_b, dsem_a, dsem_b):
        cid = lax.axis_index("c")
        tid = lax.axis_index("s")
        wid = tid * 2 + cid
        pltpu.sync_copy(zer_h.at[pl.ds(tid * RPT, RPT)],
                        den_sh.at[pl.ds(tid * RPT, RPT)])
        plsc.subcore_barrier()
        base = wid * pt

        bufs = {
            0: (sidx_a, didx_a, abuf_a, bbuf_a, exbuf_a, asem_a, bsem_a,
                dsem_a),
            1: (sidx_b, didx_b, abuf_b, bbuf_b, exbuf_b, asem_b, bsem_b,
                dsem_b),
        }

        def start(bi, t):
            sidx, didx, abuf, bbuf, exbuf, asem, bsem, dsem = bufs[bi]
            off = base + t * B
            pltpu.sync_copy(src_h.at[pl.ds(off, B)], sidx)
            pltpu.sync_copy(dst_h.at[pl.ds(off, B)], didx)
            pltpu.async_copy(ta_h.at[sidx], abuf, asem)
            pltpu.async_copy(tb_h.at[didx], bbuf, bsem)

        def finish(bi, t):
            sidx, didx, abuf, bbuf, exbuf, asem, bsem, dsem = bufs[bi]
            off = base + t * B
            pltpu.make_async_copy(ta_h.at[sidx], abuf, asem).wait()
            pltpu.make_async_copy(tb_h.at[didx], bbuf, bsem).wait()

            @plsc.parallel_loop(0, B, unroll=8)
            def cbody(k):
                v = abuf[k] + bbuf[k]
                v = jnp.maximum(v, 0.2 * v)
                exbuf[k] = jnp.exp(v)
            pltpu.sync_copy(exbuf, ex_h.at[pl.ds(off, B)])
            pltpu.sync_copy(exbuf, den_sh.at[didx], add=True)

        start(0, 0)

        def pair_body(p, carry):
            t0 = 2 * p
            start(1, t0 + 1)
            finish(0, t0)
            start(0, t0 + 2)
            finish(1, t0 + 1)
            return carry

        lax.fori_loop(0, (nb - 1) // 2, pair_body, 0)
        finish(0, nb - 1)
        plsc.subcore_barrier()
        pltpu.sync_copy(den_sh.at[pl.ds(tid * RPT, RPT)],
                        denp_h.at[cid, pl.ds(tid * RPT, RPT)])

    return exdenom


def _make_accum(e2p, nheads, nb0):
    nbt = e2p // (NSUB * B)
    nb1 = nbt - nb0
    mesh = plsc.VectorSubcoreMesh(core_axis_name="c", subcore_axis_name="s")

    @functools.partial(
        pl.kernel, mesh=mesh,
        compiler_params=pltpu.CompilerParams(use_tc_tiling_on_sc=False),
        out_type=jax.ShapeDtypeStruct((2, nheads, NACC, D), jnp.float32),
        scratch_types=[
            pltpu.VMEM((max(nb0, nb1) * B,), jnp.int32),
            pltpu.VMEM((B,), jnp.int32),
            pltpu.VMEM((B,), jnp.int32),
            pltpu.VMEM((B,), jnp.int32),
            pltpu.VMEM((B,), jnp.int32),
            pltpu.VMEM((B, 16), jnp.float32),
            pltpu.VMEM((B, 16), jnp.float32),
            pltpu.VMEM((B, D), jnp.float32),
            pltpu.VMEM((B, D), jnp.float32),
            pltpu.VMEM_SHARED((NACC, D), jnp.float32),
            pltpu.SemaphoreType.DMA,
            pltpu.SemaphoreType.DMA,
            pltpu.SemaphoreType.DMA,
            pltpu.SemaphoreType.DMA,
            pltpu.SemaphoreType.DMA,
            pltpu.SemaphoreType.DMA,
            pltpu.SemaphoreType.DMA,
            pltpu.SemaphoreType.DMA,
        ],
    )
    def accum(src_h, dst_h, ex_h, hflat_h, zer_h, acc_h,
              sidx_all, didx_a, didx_b, gidx_a, gidx_b, exb_a, exb_b,
              rows_a, rows_b, acc_sh,
              gsem_a, gsem_b, xsem_a, xsem_b, dsem_a, dsem_b,
              ssem_a, ssem_b):
        cid = lax.axis_index("c")
        tid = lax.axis_index("s")
        nbc = jnp.where(cid == 0, nb0, nb1)
        base = jnp.where(cid == 0, tid * (nb0 * B),
                         NSUB * nb0 * B + tid * (nb1 * B))
        pltpu.sync_copy(src_h.at[pl.ds(base, max(nb0, nb1) * B)], sidx_all)

        bufs = {
            0: (gidx_a, didx_a, exb_a, rows_a, gsem_a, xsem_a, dsem_a,
                ssem_a),
            1: (gidx_b, didx_b, exb_b, rows_b, gsem_b, xsem_b, dsem_b,
                ssem_b),
        }

        def start(bi, t, hh):
            gidx, didx, exb, rows, gsem, xsem, dsem, ssem = bufs[bi]
            for g in range(B // 16):
                s16 = sidx_all[pl.ds(t * B + g * 16, 16)]
                if nheads > 1:
                    gidx[pl.ds(g * 16, 16)] = s16 * nheads + hh
                else:
                    gidx[pl.ds(g * 16, 16)] = s16
            pltpu.async_copy(hflat_h.at[gidx], rows, gsem)
            pltpu.async_copy(ex_h.at[pl.ds(base + t * B, B)], exb, xsem)
            pltpu.async_copy(dst_h.at[pl.ds(base + t * B, B)], didx, dsem)

        def mul(bi, t, hh):
            gidx, didx, exb, rows, gsem, xsem, dsem, ssem = bufs[bi]
            pltpu.make_async_copy(hflat_h.at[gidx], rows, gsem).wait()
            pltpu.make_async_copy(ex_h.at[pl.ds(base + t * B, B)], exb,
                                  xsem).wait()
            pltpu.make_async_copy(dst_h.at[pl.ds(base + t * B, B)], didx,
                                  dsem).wait()

            @plsc.parallel_loop(0, B, unroll=8)
            def wbody(j):
                wl = jnp.full((16,), exb[j][hh], jnp.float32)
                for c in range(D // 16):
                    rows[j, pl.ds(c * 16, 16)] = (
                        rows[j, pl.ds(c * 16, 16)] * wl)

        def finish(bi, t, hh):
            gidx, didx, exb, rows, gsem, xsem, dsem, ssem = bufs[bi]
            mul(bi, t, hh)
            pltpu.sync_copy(rows, acc_sh.at[didx], add=True)

        for hh in range(nheads):
            pltpu.sync_copy(zer_h.at[pl.ds(tid * RPT, RPT)],
                            acc_sh.at[pl.ds(tid * RPT, RPT)])
            plsc.subcore_barrier()
            start(0, 0, hh)

            def pair_body(p, c1):
                t0 = 2 * p
                start(1, t0 + 1, hh)
                finish(0, t0, hh)
                start(0, t0 + 2, hh)
                finish(1, t0 + 1, hh)
                return c1

            lax.fori_loop(0, (nbc - 1) // 2, pair_body, 0)
            finish(0, nbc - 1, hh)
            plsc.subcore_barrier()
            pltpu.sync_copy(acc_sh.at[pl.ds(tid * RPT, RPT)],
                            acc_h.at[cid, hh, pl.ds(tid * RPT, RPT)])
            plsc.subcore_barrier()

    return accum


def kernel(x, edge_index, batch, W1, a_src1, a_dst1, b1, W2, a_src2, a_dst2,
           b2, Wfc, bfc):
    n = x.shape[0]
    e = edge_index.shape[1]
    e2 = e + n
    e2p = -(-e2 // (B * NW)) * (B * NW)
    nbt = e2p // (NSUB * B)
    nb0 = 97
    pad_extra = max(0, (2 * nb0 - nbt)) * B

    loop = jnp.arange(n, dtype=edge_index.dtype)
    src = jnp.concatenate([edge_index[0], loop,
                           jnp.zeros((e2p - e2 + pad_extra,),
                                     edge_index.dtype)])
    dst = jnp.concatenate([edge_index[1], loop,
                           jnp.full((e2p - e2 + pad_extra,), n,
                                    edge_index.dtype)])

    eye = jnp.eye(HEADS, dtype=jnp.float32)
    sel1 = jnp.concatenate(
        [jnp.repeat(eye, D, axis=0), jnp.zeros((HEADS * D, 8), jnp.float32)],
        axis=1)
    a2s = jnp.concatenate([a_src2.T, jnp.zeros((D, 15), jnp.float32)], axis=1)
    a2d = jnp.concatenate([a_dst2.T, jnp.zeros((D, 15), jnp.float32)], axis=1)

    zer_den = jnp.zeros((NACC, 16), jnp.float32)
    zer_acc = jnp.zeros((NACC, D), jnp.float32)
    padrows = jnp.zeros((NACC - n, 16), jnp.float32)

    h1, ta1, tb1 = _mm_alpha(x, W1, a_src1.reshape(1, -1),
                             a_dst1.reshape(1, -1), sel1)
    ta1p = jnp.concatenate([ta1, padrows])
    tb1p = jnp.concatenate([tb1, padrows])
    ex1, denp1 = _make_exdenom(e2p)(src, dst, ta1p, tb1p, zer_den)
    acc1 = _make_accum(e2p, HEADS, nb0)(src, dst, ex1,
                                        h1.reshape(n * HEADS, D), zer_acc)

    h2, ta2, tb2 = _combine2(acc1, denp1, b1.reshape(HEADS, D),
                             W2.reshape(HEADS, D, D), a2s, a2d)
    ta2p = jnp.concatenate([ta2, padrows])
    tb2p = jnp.concatenate([tb2, padrows])
    ex2, denp2 = _make_exdenom(e2p)(src, dst, ta2p, tb2p, zer_den)
    acc2 = _make_accum(e2p, 1, nb0)(src, dst, ex2, h2, zer_acc)

    out2 = _combine1(acc2, denp2, b2.reshape(1, D))

    return _pool_fc(out2, batch.reshape(n, 1).astype(jnp.int32), Wfc,
                    bfc.reshape(1, D))


# speedup vs baseline: 1.1850x; 1.0112x over previous
import functools

import jax
import jax.numpy as jnp
from jax import lax
from jax.experimental import pallas as pl
from jax.experimental.pallas import tpu as pltpu
from jax.experimental.pallas import tpu_sc as plsc

N = 10000
HEADS = 8
D = 128
G = 64
NW = 32
NSUB = 16
B = 128
NACC = 10112
RPT = NACC // NSUB
R = 400


def _mm_alpha_body(x_ref, w_ref, asf_ref, adf_ref, sel_ref, h_ref, ta_ref, tb_ref):
    h = jnp.dot(x_ref[...], w_ref[...], preferred_element_type=jnp.float32)
    h_ref[...] = h
    ta_ref[...] = jnp.dot(h * asf_ref[...], sel_ref[...],
                          preferred_element_type=jnp.float32)
    tb_ref[...] = jnp.dot(h * adf_ref[...], sel_ref[...],
                          preferred_element_type=jnp.float32)


def _combine2_body(acc_ref, den_ref, b1_ref, w2_ref, a2s_ref, a2d_ref,
                   h2_ref, ta_ref, tb_ref):
    den = den_ref[0] + den_ref[1]
    h2 = jnp.zeros((R, D), jnp.float32)
    for h in range(HEADS):
        a = acc_ref[0, h] + acc_ref[1, h]
        x2 = jnp.maximum(a / (den[:, h:h + 1] + 1e-16) + b1_ref[h][None, :], 0.0)
        h2 = h2 + jnp.dot(x2, w2_ref[h], preferred_element_type=jnp.float32)
    h2_ref[...] = h2
    ta_ref[...] = jnp.dot(h2, a2s_ref[...], preferred_element_type=jnp.float32)
    tb_ref[...] = jnp.dot(h2, a2d_ref[...], preferred_element_type=jnp.float32)


def _combine1_body(acc_ref, den_ref, b2_ref, out_ref):
    den = den_ref[0] + den_ref[1]
    a = acc_ref[0, 0] + acc_ref[1, 0]
    out_ref[...] = jnp.maximum(a / (den[:, 0:1] + 1e-16) + b2_ref[...], 0.0)


def _pool_fc_body(x_ref, bt_ref, wfc_ref, bfc_ref, out_ref):
    pid = pl.program_id(0)
    x = x_ref[...]
    bt = bt_ref[...]
    rows = []
    for gl in range(8):
        g = pid * 8 + gl
        m = jnp.where(bt == g, x, -jnp.inf)
        rows.append(jnp.max(m, axis=0, keepdims=True))
    pooled = jnp.concatenate(rows, axis=0)
    pooled = jnp.where(pooled == -jnp.inf, 0.0, pooled)
    out_ref[...] = jnp.maximum(
        jnp.dot(pooled, wfc_ref[...], preferred_element_type=jnp.float32)
        + bfc_ref[...], 0.0)


def _mm_alpha(x, w, asf, adf, sel):
    n = x.shape[0]
    grid = n // R
    return pl.pallas_call(
        _mm_alpha_body,
        grid=(grid,),
        in_specs=[
            pl.BlockSpec((R, x.shape[1]), lambda i: (i, 0)),
            pl.BlockSpec(w.shape, lambda i: (0, 0)),
            pl.BlockSpec(asf.shape, lambda i: (0, 0)),
            pl.BlockSpec(adf.shape, lambda i: (0, 0)),
            pl.BlockSpec(sel.shape, lambda i: (0, 0)),
        ],
        out_specs=[
            pl.BlockSpec((R, w.shape[1]), lambda i: (i, 0)),
            pl.BlockSpec((R, 16), lambda i: (i, 0)),
            pl.BlockSpec((R, 16), lambda i: (i, 0)),
        ],
        out_shape=[
            jax.ShapeDtypeStruct((n, w.shape[1]), jnp.float32),
            jax.ShapeDtypeStruct((n, 16), jnp.float32),
            jax.ShapeDtypeStruct((n, 16), jnp.float32),
        ],
    )(x, w, asf, adf, sel)


def _combine2(acc, den, b1r, w2r, a2s, a2d):
    grid = N // R
    return pl.pallas_call(
        _combine2_body,
        grid=(grid,),
        in_specs=[
            pl.BlockSpec((2, HEADS, R, D), lambda i: (0, 0, i, 0)),
            pl.BlockSpec((2, R, 16), lambda i: (0, i, 0)),
            pl.BlockSpec((HEADS, D), lambda i: (0, 0)),
            pl.BlockSpec((HEADS, D, D), lambda i: (0, 0, 0)),
            pl.BlockSpec((D, 16), lambda i: (0, 0)),
            pl.BlockSpec((D, 16), lambda i: (0, 0)),
        ],
        out_specs=[
            pl.BlockSpec((R, D), lambda i: (i, 0)),
            pl.BlockSpec((R, 16), lambda i: (i, 0)),
            pl.BlockSpec((R, 16), lambda i: (i, 0)),
        ],
        out_shape=[
            jax.ShapeDtypeStruct((N, D), jnp.float32),
            jax.ShapeDtypeStruct((N, 16), jnp.float32),
            jax.ShapeDtypeStruct((N, 16), jnp.float32),
        ],
    )(acc, den, b1r, w2r, a2s, a2d)


def _combine1(acc, den, b2r):
    grid = N // R
    return pl.pallas_call(
        _combine1_body,
        grid=(grid,),
        in_specs=[
            pl.BlockSpec((2, 1, R, D), lambda i: (0, 0, i, 0)),
            pl.BlockSpec((2, R, 16), lambda i: (0, i, 0)),
            pl.BlockSpec((1, D), lambda i: (0, 0)),
        ],
        out_specs=pl.BlockSpec((R, D), lambda i: (i, 0)),
        out_shape=jax.ShapeDtypeStruct((N, D), jnp.float32),
    )(acc, den, b2r)


def _pool_fc(x, bt, wfc, bfcr):
    return pl.pallas_call(
        _pool_fc_body,
        grid=(G // 8,),
        in_specs=[
            pl.BlockSpec((N, D), lambda i: (0, 0)),
            pl.BlockSpec((N, 1), lambda i: (0, 0)),
            pl.BlockSpec((D, D), lambda i: (0, 0)),
            pl.BlockSpec((1, D), lambda i: (0, 0)),
        ],
        out_specs=pl.BlockSpec((8, D), lambda i: (i, 0)),
        out_shape=jax.ShapeDtypeStruct((G, D), jnp.float32),
    )(x, bt, wfc, bfcr)


def _make_exdenom(e2p):
    pt = e2p // NW
    nb = pt // B
    mesh = plsc.VectorSubcoreMesh(core_axis_name="c", subcore_axis_name="s")

    @functools.partial(
        pl.kernel, mesh=mesh,
        compiler_params=pltpu.CompilerParams(use_tc_tiling_on_sc=False),
        out_type=[
            jax.ShapeDtypeStruct((e2p, 16), jnp.float32),
            jax.ShapeDtypeStruct((2, NACC, 16), jnp.float32),
        ],
        scratch_types=[
            pltpu.VMEM((B,), jnp.int32),
            pltpu.VMEM((B,), jnp.int32),
            pltpu.VMEM((B,), jnp.int32),
            pltpu.VMEM((B,), jnp.int32),
            pltpu.VMEM((B, 16), jnp.float32),
            pltpu.VMEM((B, 16), jnp.float32),
            pltpu.VMEM((B, 16), jnp.float32),
            pltpu.VMEM((B, 16), jnp.float32),
            pltpu.VMEM((B, 16), jnp.float32),
            pltpu.VMEM((B, 16), jnp.float32),
            pltpu.VMEM_SHARED((NACC, 16), jnp.float32),
            pltpu.SemaphoreType.DMA,
            pltpu.SemaphoreType.DMA,
            pltpu.SemaphoreType.DMA,
            pltpu.SemaphoreType.DMA,
            pltpu.SemaphoreType.DMA,
            pltpu.SemaphoreType.DMA,
        ],
    )
    def exdenom(src_h, dst_h, ta_h, tb_h, zer_h, ex_h, denp_h,
                sidx_a, sidx_b, didx_a, didx_b, abuf_a, abuf_b,
                bbuf_a, bbuf_b, exbuf_a, exbuf_b, den_sh,
                asem_a, asem_b, bsem_a, bsem_b, dsem_a, dsem_b):
        cid = lax.axis_index("c")
        tid = lax.axis_index("s")
        wid = tid * 2 + cid
        pltpu.sync_copy(zer_h.at[pl.ds(tid * RPT, RPT)],
                        den_sh.at[pl.ds(tid * RPT, RPT)])
        plsc.subcore_barrier()
        base = wid * pt

        bufs = {
            0: (sidx_a, didx_a, abuf_a, bbuf_a, exbuf_a, asem_a, bsem_a,
                dsem_a),
            1: (sidx_b, didx_b, abuf_b, bbuf_b, exbuf_b, asem_b, bsem_b,
                dsem_b),
        }

        def start(bi, t):
            sidx, didx, abuf, bbuf, exbuf, asem, bsem, dsem = bufs[bi]
            off = base + t * B
            pltpu.sync_copy(src_h.at[pl.ds(off, B)], sidx)
            pltpu.sync_copy(dst_h.at[pl.ds(off, B)], didx)
            pltpu.async_copy(ta_h.at[sidx], abuf, asem)
            pltpu.async_copy(tb_h.at[didx], bbuf, bsem)

        def finish(bi, t):
            sidx, didx, abuf, bbuf, exbuf, asem, bsem, dsem = bufs[bi]
            off = base + t * B
            pltpu.make_async_copy(ta_h.at[sidx], abuf, asem).wait()
            pltpu.make_async_copy(tb_h.at[didx], bbuf, bsem).wait()

            @plsc.parallel_loop(0, B, unroll=8)
            def cbody(k):
                v = abuf[k] + bbuf[k]
                v = jnp.maximum(v, 0.2 * v)
                exbuf[k] = jnp.exp(v)
            pltpu.sync_copy(exbuf, ex_h.at[pl.ds(off, B)])
            pltpu.sync_copy(exbuf, den_sh.at[didx], add=True)

        start(0, 0)

        def pair_body(p, carry):
            t0 = 2 * p
            start(1, t0 + 1)
            finish(0, t0)
            start(0, t0 + 2)
            finish(1, t0 + 1)
            return carry

        lax.fori_loop(0, (nb - 1) // 2, pair_body, 0)
        finish(0, nb - 1)
        plsc.subcore_barrier()
        pltpu.sync_copy(den_sh.at[pl.ds(tid * RPT, RPT)],
                        denp_h.at[cid, pl.ds(tid * RPT, RPT)])

    return exdenom


def _make_accum(e2p, nheads, nb0):
    nbt = e2p // (NSUB * B)
    nb1 = nbt - nb0
    mesh = plsc.VectorSubcoreMesh(core_axis_name="c", subcore_axis_name="s")

    @functools.partial(
        pl.kernel, mesh=mesh,
        compiler_params=pltpu.CompilerParams(use_tc_tiling_on_sc=False),
        out_type=jax.ShapeDtypeStruct((2, nheads, NACC, D), jnp.float32),
        scratch_types=[
            pltpu.VMEM((max(nb0, nb1) * B,), jnp.int32),
            pltpu.VMEM((B,), jnp.int32),
            pltpu.VMEM((B,), jnp.int32),
            pltpu.VMEM((B,), jnp.int32),
            pltpu.VMEM((B,), jnp.int32),
            pltpu.VMEM((B, 16), jnp.float32),
            pltpu.VMEM((B, 16), jnp.float32),
            pltpu.VMEM((B, D), jnp.float32),
            pltpu.VMEM((B, D), jnp.float32),
            pltpu.VMEM_SHARED((NACC, D), jnp.float32),
            pltpu.SemaphoreType.DMA,
            pltpu.SemaphoreType.DMA,
            pltpu.SemaphoreType.DMA,
            pltpu.SemaphoreType.DMA,
            pltpu.SemaphoreType.DMA,
            pltpu.SemaphoreType.DMA,
            pltpu.SemaphoreType.DMA,
            pltpu.SemaphoreType.DMA,
        ],
    )
    def accum(src_h, dst_h, ex_h, hflat_h, zer_h, acc_h,
              sidx_all, didx_a, didx_b, gidx_a, gidx_b, exb_a, exb_b,
              rows_a, rows_b, acc_sh,
              gsem_a, gsem_b, xsem_a, xsem_b, dsem_a, dsem_b,
              ssem_a, ssem_b):
        cid = lax.axis_index("c")
        tid = lax.axis_index("s")
        nbc = jnp.where(cid == 0, nb0, nb1)
        base = jnp.where(cid == 0, tid * (nb0 * B),
                         NSUB * nb0 * B + tid * (nb1 * B))
        pltpu.sync_copy(src_h.at[pl.ds(base, max(nb0, nb1) * B)], sidx_all)

        bufs = {
            0: (gidx_a, didx_a, exb_a, rows_a, gsem_a, xsem_a, dsem_a,
                ssem_a),
            1: (gidx_b, didx_b, exb_b, rows_b, gsem_b, xsem_b, dsem_b,
                ssem_b),
        }

        def start(bi, t, hh):
            gidx, didx, exb, rows, gsem, xsem, dsem, ssem = bufs[bi]
            for g in range(B // 16):
                s16 = sidx_all[pl.ds(t * B + g * 16, 16)]
                if nheads > 1:
                    gidx[pl.ds(g * 16, 16)] = s16 * nheads + hh
                else:
                    gidx[pl.ds(g * 16, 16)] = s16
            pltpu.async_copy(hflat_h.at[gidx], rows, gsem)
            pltpu.async_copy(ex_h.at[pl.ds(base + t * B, B)], exb, xsem)
            pltpu.async_copy(dst_h.at[pl.ds(base + t * B, B)], didx, dsem)

        def mul(bi, t, hh):
            gidx, didx, exb, rows, gsem, xsem, dsem, ssem = bufs[bi]
            pltpu.make_async_copy(hflat_h.at[gidx], rows, gsem).wait()
            pltpu.make_async_copy(ex_h.at[pl.ds(base + t * B, B)], exb,
                                  xsem).wait()
            pltpu.make_async_copy(dst_h.at[pl.ds(base + t * B, B)], didx,
                                  dsem).wait()

            @plsc.parallel_loop(0, B, unroll=8)
            def wbody(j):
                wl = jnp.full((16,), exb[j][hh], jnp.float32)
                for c in range(D // 16):
                    rows[j, pl.ds(c * 16, 16)] = (
                        rows[j, pl.ds(c * 16, 16)] * wl)

        def finish(bi, t, hh):
            gidx, didx, exb, rows, gsem, xsem, dsem, ssem = bufs[bi]
            mul(bi, t, hh)
            pltpu.sync_copy(rows, acc_sh.at[didx], add=True)

        for hh in range(nheads):
            pltpu.sync_copy(zer_h.at[pl.ds(tid * RPT, RPT)],
                            acc_sh.at[pl.ds(tid * RPT, RPT)])
            plsc.subcore_barrier()
            start(0, 0, hh)

            def pair_body(p, c1):
                t0 = 2 * p
                start(1, t0 + 1, hh)
                finish(0, t0, hh)
                start(0, t0 + 2, hh)
                finish(1, t0 + 1, hh)
                return c1

            lax.fori_loop(0, (nbc - 1) // 2, pair_body, 0)
            finish(0, nbc - 1, hh)
            plsc.subcore_barrier()
            pltpu.sync_copy(acc_sh.at[pl.ds(tid * RPT, RPT)],
                            acc_h.at[cid, hh, pl.ds(tid * RPT, RPT)])
            plsc.subcore_barrier()

    return accum


def kernel(x, edge_index, batch, W1, a_src1, a_dst1, b1, W2, a_src2, a_dst2,
           b2, Wfc, bfc):
    n = x.shape[0]
    e = edge_index.shape[1]
    e2 = e + n
    e2p = -(-e2 // (B * NW)) * (B * NW)
    nbt = e2p // (NSUB * B)
    nb0 = 99
    pad_extra = max(0, (2 * nb0 - nbt)) * B

    loop = jnp.arange(n, dtype=edge_index.dtype)
    src = jnp.concatenate([edge_index[0], loop,
                           jnp.zeros((e2p - e2 + pad_extra,),
                                     edge_index.dtype)])
    dst = jnp.concatenate([edge_index[1], loop,
                           jnp.full((e2p - e2 + pad_extra,), n,
                                    edge_index.dtype)])

    eye = jnp.eye(HEADS, dtype=jnp.float32)
    sel1 = jnp.concatenate(
        [jnp.repeat(eye, D, axis=0), jnp.zeros((HEADS * D, 8), jnp.float32)],
        axis=1)
    a2s = jnp.concatenate([a_src2.T, jnp.zeros((D, 15), jnp.float32)], axis=1)
    a2d = jnp.concatenate([a_dst2.T, jnp.zeros((D, 15), jnp.float32)], axis=1)

    zer_den = jnp.zeros((NACC, 16), jnp.float32)
    zer_acc = jnp.zeros((NACC, D), jnp.float32)
    padrows = jnp.zeros((NACC - n, 16), jnp.float32)

    h1, ta1, tb1 = _mm_alpha(x, W1, a_src1.reshape(1, -1),
                             a_dst1.reshape(1, -1), sel1)
    ta1p = jnp.concatenate([ta1, padrows])
    tb1p = jnp.concatenate([tb1, padrows])
    ex1, denp1 = _make_exdenom(e2p)(src, dst, ta1p, tb1p, zer_den)
    acc1 = _make_accum(e2p, HEADS, nb0)(src, dst, ex1,
                                        h1.reshape(n * HEADS, D), zer_acc)

    h2, ta2, tb2 = _combine2(acc1, denp1, b1.reshape(HEADS, D),
                             W2.reshape(HEADS, D, D), a2s, a2d)
    ta2p = jnp.concatenate([ta2, padrows])
    tb2p = jnp.concatenate([tb2, padrows])
    ex2, denp2 = _make_exdenom(e2p)(src, dst, ta2p, tb2p, zer_den)
    acc2 = _make_accum(e2p, 1, nb0)(src, dst, ex2, h2, zer_acc)

    out2 = _combine1(acc2, denp2, b2.reshape(1, D))

    return _pool_fc(out2, batch.reshape(n, 1).astype(jnp.int32), Wfc,
                    bfc.reshape(1, D))


# speedup vs baseline: 1.1856x; 1.0005x over previous
import functools

import jax
import jax.numpy as jnp
from jax import lax
from jax.experimental import pallas as pl
from jax.experimental.pallas import tpu as pltpu
from jax.experimental.pallas import tpu_sc as plsc

N = 10000
HEADS = 8
D = 128
G = 64
NW = 32
NSUB = 16
B = 128
NACC = 10112
RPT = NACC // NSUB
R = 400


def _mm_alpha_body(x_ref, w_ref, asf_ref, adf_ref, sel_ref, h_ref, ta_ref, tb_ref):
    h = jnp.dot(x_ref[...], w_ref[...], preferred_element_type=jnp.float32)
    h_ref[...] = h
    ta_ref[...] = jnp.dot(h * asf_ref[...], sel_ref[...],
                          preferred_element_type=jnp.float32)
    tb_ref[...] = jnp.dot(h * adf_ref[...], sel_ref[...],
                          preferred_element_type=jnp.float32)


def _combine2_body(acc_ref, den_ref, b1_ref, w2_ref, a2s_ref, a2d_ref,
                   h2_ref, ta_ref, tb_ref):
    den = den_ref[0] + den_ref[1]
    h2 = jnp.zeros((R, D), jnp.float32)
    for h in range(HEADS):
        a = acc_ref[0, h] + acc_ref[1, h]
        x2 = jnp.maximum(a / (den[:, h:h + 1] + 1e-16) + b1_ref[h][None, :], 0.0)
        h2 = h2 + jnp.dot(x2, w2_ref[h], preferred_element_type=jnp.float32)
    h2_ref[...] = h2
    ta_ref[...] = jnp.dot(h2, a2s_ref[...], preferred_element_type=jnp.float32)
    tb_ref[...] = jnp.dot(h2, a2d_ref[...], preferred_element_type=jnp.float32)


def _combine1_body(acc_ref, den_ref, b2_ref, out_ref):
    den = den_ref[0] + den_ref[1]
    a = acc_ref[0, 0] + acc_ref[1, 0]
    out_ref[...] = jnp.maximum(a / (den[:, 0:1] + 1e-16) + b2_ref[...], 0.0)


def _pool_fc_body(x_ref, bt_ref, wfc_ref, bfc_ref, out_ref):
    pid = pl.program_id(0)
    x = x_ref[...]
    bt = bt_ref[...]
    rows = []
    for gl in range(8):
        g = pid * 8 + gl
        m = jnp.where(bt == g, x, -jnp.inf)
        rows.append(jnp.max(m, axis=0, keepdims=True))
    pooled = jnp.concatenate(rows, axis=0)
    pooled = jnp.where(pooled == -jnp.inf, 0.0, pooled)
    out_ref[...] = jnp.maximum(
        jnp.dot(pooled, wfc_ref[...], preferred_element_type=jnp.float32)
        + bfc_ref[...], 0.0)


def _mm_alpha(x, w, asf, adf, sel):
    n = x.shape[0]
    grid = n // R
    return pl.pallas_call(
        _mm_alpha_body,
        grid=(grid,),
        in_specs=[
            pl.BlockSpec((R, x.shape[1]), lambda i: (i, 0)),
            pl.BlockSpec(w.shape, lambda i: (0, 0)),
            pl.BlockSpec(asf.shape, lambda i: (0, 0)),
            pl.BlockSpec(adf.shape, lambda i: (0, 0)),
            pl.BlockSpec(sel.shape, lambda i: (0, 0)),
        ],
        out_specs=[
            pl.BlockSpec((R, w.shape[1]), lambda i: (i, 0)),
            pl.BlockSpec((R, 16), lambda i: (i, 0)),
            pl.BlockSpec((R, 16), lambda i: (i, 0)),
        ],
        out_shape=[
            jax.ShapeDtypeStruct((n, w.shape[1]), jnp.float32),
            jax.ShapeDtypeStruct((n, 16), jnp.float32),
            jax.ShapeDtypeStruct((n, 16), jnp.float32),
        ],
    )(x, w, asf, adf, sel)


def _combine2(acc, den, b1r, w2r, a2s, a2d):
    grid = N // R
    return pl.pallas_call(
        _combine2_body,
        grid=(grid,),
        in_specs=[
            pl.BlockSpec((2, HEADS, R, D), lambda i: (0, 0, i, 0)),
            pl.BlockSpec((2, R, 16), lambda i: (0, i, 0)),
            pl.BlockSpec((HEADS, D), lambda i: (0, 0)),
            pl.BlockSpec((HEADS, D, D), lambda i: (0, 0, 0)),
            pl.BlockSpec((D, 16), lambda i: (0, 0)),
            pl.BlockSpec((D, 16), lambda i: (0, 0)),
        ],
        out_specs=[
            pl.BlockSpec((R, D), lambda i: (i, 0)),
            pl.BlockSpec((R, 16), lambda i: (i, 0)),
            pl.BlockSpec((R, 16), lambda i: (i, 0)),
        ],
        out_shape=[
            jax.ShapeDtypeStruct((N, D), jnp.float32),
            jax.ShapeDtypeStruct((N, 16), jnp.float32),
            jax.ShapeDtypeStruct((N, 16), jnp.float32),
        ],
    )(acc, den, b1r, w2r, a2s, a2d)


def _combine1(acc, den, b2r):
    grid = N // R
    return pl.pallas_call(
        _combine1_body,
        grid=(grid,),
        in_specs=[
            pl.BlockSpec((2, 1, R, D), lambda i: (0, 0, i, 0)),
            pl.BlockSpec((2, R, 16), lambda i: (0, i, 0)),
            pl.BlockSpec((1, D), lambda i: (0, 0)),
        ],
        out_specs=pl.BlockSpec((R, D), lambda i: (i, 0)),
        out_shape=jax.ShapeDtypeStruct((N, D), jnp.float32),
    )(acc, den, b2r)


def _pool_fc(x, bt, wfc, bfcr):
    return pl.pallas_call(
        _pool_fc_body,
        grid=(G // 8,),
        in_specs=[
            pl.BlockSpec((N, D), lambda i: (0, 0)),
            pl.BlockSpec((N, 1), lambda i: (0, 0)),
            pl.BlockSpec((D, D), lambda i: (0, 0)),
            pl.BlockSpec((1, D), lambda i: (0, 0)),
        ],
        out_specs=pl.BlockSpec((8, D), lambda i: (i, 0)),
        out_shape=jax.ShapeDtypeStruct((G, D), jnp.float32),
    )(x, bt, wfc, bfcr)


def _make_exdenom(e2p):
    pt = e2p // NW
    nb = pt // B
    mesh = plsc.VectorSubcoreMesh(core_axis_name="c", subcore_axis_name="s")

    @functools.partial(
        pl.kernel, mesh=mesh,
        compiler_params=pltpu.CompilerParams(use_tc_tiling_on_sc=False),
        out_type=[
            jax.ShapeDtypeStruct((e2p, 16), jnp.float32),
            jax.ShapeDtypeStruct((2, NACC, 16), jnp.float32),
        ],
        scratch_types=[
            pltpu.VMEM((B,), jnp.int32),
            pltpu.VMEM((B,), jnp.int32),
            pltpu.VMEM((B,), jnp.int32),
            pltpu.VMEM((B,), jnp.int32),
            pltpu.VMEM((B, 16), jnp.float32),
            pltpu.VMEM((B, 16), jnp.float32),
            pltpu.VMEM((B, 16), jnp.float32),
            pltpu.VMEM((B, 16), jnp.float32),
            pltpu.VMEM((B, 16), jnp.float32),
            pltpu.VMEM((B, 16), jnp.float32),
            pltpu.VMEM_SHARED((NACC, 16), jnp.float32),
            pltpu.SemaphoreType.DMA,
            pltpu.SemaphoreType.DMA,
            pltpu.SemaphoreType.DMA,
            pltpu.SemaphoreType.DMA,
            pltpu.SemaphoreType.DMA,
            pltpu.SemaphoreType.DMA,
        ],
    )
    def exdenom(src_h, dst_h, ta_h, tb_h, zer_h, ex_h, denp_h,
                sidx_a, sidx_b, didx_a, didx_b, abuf_a, abuf_b,
                bbuf_a, bbuf_b, exbuf_a, exbuf_b, den_sh,
                asem_a, asem_b, bsem_a, bsem_b, dsem_a, dsem_b):
        cid = lax.axis_index("c")
        tid = lax.axis_index("s")
        wid = tid * 2 + cid
        pltpu.sync_copy(zer_h.at[pl.ds(tid * RPT, RPT)],
                        den_sh.at[pl.ds(tid * RPT, RPT)])
        plsc.subcore_barrier()
        base = wid * pt

        bufs = {
            0: (sidx_a, didx_a, abuf_a, bbuf_a, exbuf_a, asem_a, bsem_a,
                dsem_a),
            1: (sidx_b, didx_b, abuf_b, bbuf_b, exbuf_b, asem_b, bsem_b,
                dsem_b),
        }

        def start(bi, t):
            sidx, didx, abuf, bbuf, exbuf, asem, bsem, dsem = bufs[bi]
            off = base + t * B
            pltpu.sync_copy(src_h.at[pl.ds(off, B)], sidx)
            pltpu.sync_copy(dst_h.at[pl.ds(off, B)], didx)
            pltpu.async_copy(ta_h.at[sidx], abuf, asem)
            pltpu.async_copy(tb_h.at[didx], bbuf, bsem)

        def finish(bi, t):
            sidx, didx, abuf, bbuf, exbuf, asem, bsem, dsem = bufs[bi]
            off = base + t * B
            pltpu.make_async_copy(ta_h.at[sidx], abuf, asem).wait()
            pltpu.make_async_copy(tb_h.at[didx], bbuf, bsem).wait()

            @plsc.parallel_loop(0, B, unroll=8)
            def cbody(k):
                v = abuf[k] + bbuf[k]
                v = jnp.maximum(v, 0.2 * v)
                exbuf[k] = jnp.exp(v)
            pltpu.sync_copy(exbuf, ex_h.at[pl.ds(off, B)])
            pltpu.sync_copy(exbuf, den_sh.at[didx], add=True)

        start(0, 0)

        def pair_body(p, carry):
            t0 = 2 * p
            start(1, t0 + 1)
            finish(0, t0)
            start(0, t0 + 2)
            finish(1, t0 + 1)
            return carry

        lax.fori_loop(0, (nb - 1) // 2, pair_body, 0)
        finish(0, nb - 1)
        plsc.subcore_barrier()
        pltpu.sync_copy(den_sh.at[pl.ds(tid * RPT, RPT)],
                        denp_h.at[cid, pl.ds(tid * RPT, RPT)])

    return exdenom


def _make_accum(e2p, nheads, nb0):
    nbt = e2p // (NSUB * B)
    nb1 = nbt - nb0
    mesh = plsc.VectorSubcoreMesh(core_axis_name="c", subcore_axis_name="s")

    @functools.partial(
        pl.kernel, mesh=mesh,
        compiler_params=pltpu.CompilerParams(use_tc_tiling_on_sc=False),
        out_type=jax.ShapeDtypeStruct((2, nheads, NACC, D), jnp.float32),
        scratch_types=[
            pltpu.VMEM((max(nb0, nb1) * B,), jnp.int32),
            pltpu.VMEM((B,), jnp.int32),
            pltpu.VMEM((B,), jnp.int32),
            pltpu.VMEM((B,), jnp.int32),
            pltpu.VMEM((B,), jnp.int32),
            pltpu.VMEM((B, 16), jnp.float32),
            pltpu.VMEM((B, 16), jnp.float32),
            pltpu.VMEM((B, D), jnp.float32),
            pltpu.VMEM((B, D), jnp.float32),
            pltpu.VMEM_SHARED((NACC, D), jnp.float32),
            pltpu.SemaphoreType.DMA,
            pltpu.SemaphoreType.DMA,
            pltpu.SemaphoreType.DMA,
            pltpu.SemaphoreType.DMA,
            pltpu.SemaphoreType.DMA,
            pltpu.SemaphoreType.DMA,
        ],
    )
    def accum(src_h, dst_h, ex_h, hflat_h, zer_h, acc_h,
              sidx_all, didx_a, didx_b, gidx_a, gidx_b, exb_a, exb_b,
              rows_a, rows_b, acc_sh,
              gsem_a, gsem_b, xsem_a, xsem_b, dsem_a, dsem_b):
        cid = lax.axis_index("c")
        tid = lax.axis_index("s")
        nbc = jnp.where(cid == 0, nb0, nb1)
        base = jnp.where(cid == 0, tid * (nb0 * B),
                         NSUB * nb0 * B + tid * (nb1 * B))
        pltpu.sync_copy(src_h.at[pl.ds(base, max(nb0, nb1) * B)], sidx_all)

        bufs = {
            0: (gidx_a, didx_a, exb_a, rows_a, gsem_a, xsem_a, dsem_a),
            1: (gidx_b, didx_b, exb_b, rows_b, gsem_b, xsem_b, dsem_b),
        }

        def start(bi, t, hh):
            gidx, didx, exb, rows, gsem, xsem, dsem = bufs[bi]
            for g in range(B // 16):
                s16 = sidx_all[pl.ds(t * B + g * 16, 16)]
                if nheads > 1:
                    gidx[pl.ds(g * 16, 16)] = s16 * nheads + hh
                else:
                    gidx[pl.ds(g * 16, 16)] = s16
            pltpu.async_copy(hflat_h.at[gidx], rows, gsem)
            pltpu.async_copy(ex_h.at[pl.ds(base + t * B, B)], exb, xsem)
            pltpu.async_copy(dst_h.at[pl.ds(base + t * B, B)], didx, dsem)

        def mul(bi, t, hh):
            gidx, didx, exb, rows, gsem, xsem, dsem = bufs[bi]
            pltpu.make_async_copy(hflat_h.at[gidx], rows, gsem).wait()
            pltpu.make_async_copy(ex_h.at[pl.ds(base + t * B, B)], exb,
                                  xsem).wait()
            pltpu.make_async_copy(dst_h.at[pl.ds(base + t * B, B)], didx,
                                  dsem).wait()

            @plsc.parallel_loop(0, B, unroll=8)
            def wbody(j):
                wl = jnp.full((16,), exb[j][hh], jnp.float32)
                for c in range(D // 16):
                    rows[j, pl.ds(c * 16, 16)] = (
                        rows[j, pl.ds(c * 16, 16)] * wl)

        def finish(bi, t, hh):
            gidx, didx, exb, rows, gsem, xsem, dsem = bufs[bi]
            mul(bi, t, hh)
            pltpu.sync_copy(rows, acc_sh.at[didx], add=True)

        for hh in range(nheads):
            pltpu.sync_copy(zer_h.at[pl.ds(tid * RPT, RPT)],
                            acc_sh.at[pl.ds(tid * RPT, RPT)])
            plsc.subcore_barrier()
            start(0, 0, hh)

            def pair_body(p, c1):
                t0 = 2 * p
                start(1, t0 + 1, hh)
                finish(0, t0, hh)
                start(0, t0 + 2, hh)
                finish(1, t0 + 1, hh)
                return c1

            lax.fori_loop(0, (nbc - 1) // 2, pair_body, 0)
            finish(0, nbc - 1, hh)
            plsc.subcore_barrier()
            pltpu.sync_copy(acc_sh.at[pl.ds(tid * RPT, RPT)],
                            acc_h.at[cid, hh, pl.ds(tid * RPT, RPT)])
            plsc.subcore_barrier()

    return accum


def kernel(x, edge_index, batch, W1, a_src1, a_dst1, b1, W2, a_src2, a_dst2,
           b2, Wfc, bfc):
    n = x.shape[0]
    e = edge_index.shape[1]
    e2 = e + n
    e2p = -(-e2 // (B * NW)) * (B * NW)
    nbt = e2p // (NSUB * B)
    nb0 = 99
    pad_extra = max(0, (2 * nb0 - nbt)) * B

    loop = jnp.arange(n, dtype=edge_index.dtype)
    src = jnp.concatenate([edge_index[0], loop,
                           jnp.zeros((e2p - e2 + pad_extra,),
                                     edge_index.dtype)])
    dst = jnp.concatenate([edge_index[1], loop,
                           jnp.full((e2p - e2 + pad_extra,), n,
                                    edge_index.dtype)])

    eye = jnp.eye(HEADS, dtype=jnp.float32)
    sel1 = jnp.concatenate(
        [jnp.repeat(eye, D, axis=0), jnp.zeros((HEADS * D, 8), jnp.float32)],
        axis=1)
    a2s = jnp.concatenate([a_src2.T, jnp.zeros((D, 15), jnp.float32)], axis=1)
    a2d = jnp.concatenate([a_dst2.T, jnp.zeros((D, 15), jnp.float32)], axis=1)

    zer_den = jnp.zeros((NACC, 16), jnp.float32)
    zer_acc = jnp.zeros((NACC, D), jnp.float32)
    padrows = jnp.zeros((NACC - n, 16), jnp.float32)

    h1, ta1, tb1 = _mm_alpha(x, W1, a_src1.reshape(1, -1),
                             a_dst1.reshape(1, -1), sel1)
    ta1p = jnp.concatenate([ta1, padrows])
    tb1p = jnp.concatenate([tb1, padrows])
    ex1, denp1 = _make_exdenom(e2p)(src, dst, ta1p, tb1p, zer_den)
    acc1 = _make_accum(e2p, HEADS, nb0)(src, dst, ex1,
                                        h1.reshape(n * HEADS, D), zer_acc)

    h2, ta2, tb2 = _combine2(acc1, denp1, b1.reshape(HEADS, D),
                             W2.reshape(HEADS, D, D), a2s, a2d)
    ta2p = jnp.concatenate([ta2, padrows])
    tb2p = jnp.concatenate([tb2, padrows])
    ex2, denp2 = _make_exdenom(e2p)(src, dst, ta2p, tb2p, zer_den)
    acc2 = _make_accum(e2p, 1, nb0)(src, dst, ex2, h2, zer_acc)

    out2 = _combine1(acc2, denp2, b2.reshape(1, D))

    return _pool_fc(out2, batch.reshape(n, 1).astype(jnp.int32), Wfc,
                    bfc.reshape(1, D))
